# baseline (speedup 1.0000x reference)
import functools
import jax
import jax.numpy as jnp
from jax import lax
from jax.experimental import pallas as pl
from jax.experimental.pallas import tpu as pltpu, tpu_sc as plsc

V_DIM = 32
NC, NS = 2, 16
NW = NC * NS
NBUF = 2
CB = 128


@functools.lru_cache(maxsize=None)
def _make_sc_kernel(B, L, DEG_ROWS):
    assert B == NW * CB
    mesh = plsc.VectorSubcoreMesh(core_axis_name="c", subcore_axis_name="s")

    @functools.partial(
        pl.kernel,
        out_type=jax.ShapeDtypeStruct((L, V_DIM, B), jnp.float32),
        mesh=mesh,
        scratch_types=[
            pltpu.VMEM((L, CB), jnp.int32),
            pltpu.VMEM((L, CB), jnp.int32),
            pltpu.VMEM((L, CB), jnp.int32),
            pltpu.VMEM((NBUF, CB, V_DIM), jnp.float32),
            pltpu.VMEM((NBUF, V_DIM, CB), jnp.float32),
            pltpu.VMEM((DEG_ROWS, V_DIM), jnp.float32),
            [pltpu.SemaphoreType.DMA] * NBUF,
            [pltpu.SemaphoreType.DMA] * NBUF,
        ],
        compiler_params=pltpu.CompilerParams(
            use_tc_tiling_on_sc=False, needs_layout_passes=False),
    )
    def k(vidx_hbm, pos_hbm, deg_hbm, Wv, Wp, Wd, out_hbm,
          iv, ip, idg, rows, tbuf, deg_tab, sa, sw):
        wid = lax.axis_index("s") * NC + lax.axis_index("c")
        base = wid * CB
        pltpu.sync_copy(Wd, deg_tab)
        pltpu.sync_copy(vidx_hbm.at[:, pl.ds(base, CB)], iv)
        pltpu.sync_copy(pos_hbm.at[:, pl.ds(base, CB)], ip)
        pltpu.sync_copy(deg_hbm.at[:, pl.ds(base, CB)], idg)

        def deg_fill(s, p):
            buf = rows.at[p]

            def grp(g, carry):
                ixv = idg[s, pl.ds(g * 16, 16)]
                for l in range(16):
                    ix = ixv[l]
                    r = g * 16 + l
                    buf[r, pl.ds(0, 16)] = deg_tab[ix, pl.ds(0, 16)]
                    buf[r, pl.ds(16, 16)] = deg_tab[ix, pl.ds(16, 16)]
                return carry

            lax.fori_loop(0, CB // 16, grp, 0)

        def transpose(p):
            src = rows.at[p]
            dst = tbuf.at[p]

            def grp(g, carry):
                bvec = g * 16 + lax.iota(jnp.int32, 16)
                for c in range(V_DIM):
                    cvec = jnp.full((16,), c, dtype=jnp.int32)
                    v = plsc.load_gather(src, [bvec, cvec])
                    dst[c, pl.ds(g * 16, 16)] = v
                return carry

            lax.fori_loop(0, CB // 16, grp, 0)

        def fire_adds(s, p):
            dst = rows.at[p]
            pltpu.async_copy(Wv.at[iv.at[s]], dst, sa[p], add=True)
            pltpu.async_copy(Wp.at[ip.at[s]], dst, sa[p], add=True)

        def wait_adds(s, p):
            dst = rows.at[p]
            pltpu.make_async_copy(Wv.at[iv.at[s]], dst, sa[p]).wait()
            pltpu.make_async_copy(Wp.at[ip.at[s]], dst, sa[p]).wait()

        def fire_wb(s, p):
            pltpu.async_copy(tbuf.at[p], out_hbm.at[s, :, pl.ds(base, CB)],
                             sw[p])

        def wait_wb(p):
            pltpu.make_async_copy(
                tbuf.at[p], out_hbm.at[0, :, pl.ds(base, CB)], sw[p]).wait()

        def step(s, u):
            p = u % NBUF
            q = (u + NBUF - 1) % NBUF
            @pl.when(s >= NBUF)
            def _():
                wait_wb(p)

            deg_fill(s, p)
            fire_adds(s, p)
            @pl.when(s >= 1)
            def _():
                wait_adds(s - 1, q)
                transpose(q)
                fire_wb(s - 1, q)

        def round_(g, carry):
            for u in range(NBUF):
                step(g * NBUF + u, u)
            return carry

        lax.fori_loop(0, L // NBUF, round_, 0)
        p_last = (L - 1) % NBUF
        wait_adds(L - 1, p_last)
        transpose(p_last)
        fire_wb(L - 1, p_last)
        for p in range(NBUF):
            wait_wb(p)

    return k


def kernel(vidx, pos, deg, W_vidx, W_pos, W_deg):
    B, L = vidx.shape
    out_t = _make_sc_kernel(B, L, W_deg.shape[0])(
        vidx.T, pos.T, deg.T, W_vidx, W_pos, W_deg)
    return jnp.transpose(out_t, (2, 0, 1))


# speedup vs baseline: 1.4913x; 1.4913x over previous
import functools
import jax
import jax.numpy as jnp
from jax import lax
from jax.experimental import pallas as pl
from jax.experimental.pallas import tpu as pltpu, tpu_sc as plsc

V_DIM = 32
NC, NS = 2, 16
NW = NC * NS
NBUF = 2
CB = 128


@functools.lru_cache(maxsize=None)
def _make_sc_kernel(B, L, DEG_ROWS):
    assert B == NW * CB
    mesh = plsc.VectorSubcoreMesh(core_axis_name="c", subcore_axis_name="s")

    @functools.partial(
        pl.kernel,
        out_type=jax.ShapeDtypeStruct((L, V_DIM, B), jnp.float32),
        mesh=mesh,
        scratch_types=[
            pltpu.VMEM((L, CB), jnp.int32),
            pltpu.VMEM((L, CB), jnp.int32),
            pltpu.VMEM((L, CB), jnp.int32),
            pltpu.VMEM((NBUF, CB, V_DIM), jnp.float32),
            pltpu.VMEM((NBUF, V_DIM, CB), jnp.float32),
            pltpu.VMEM((DEG_ROWS, V_DIM), jnp.float32),
            [pltpu.SemaphoreType.DMA] * NBUF,
            [pltpu.SemaphoreType.DMA] * NBUF,
        ],
        compiler_params=pltpu.CompilerParams(
            use_tc_tiling_on_sc=False, needs_layout_passes=False),
    )
    def k(vidx_hbm, pos_hbm, deg_hbm, Wv, Wp, Wd, out_hbm,
          iv, ip, idg, rows, tbuf, deg_tab, sa, sw):
        wid = lax.axis_index("s") * NC + lax.axis_index("c")
        base = wid * CB
        pltpu.sync_copy(Wd, deg_tab)
        pltpu.sync_copy(vidx_hbm.at[:, pl.ds(base, CB)], iv)
        pltpu.sync_copy(pos_hbm.at[:, pl.ds(base, CB)], ip)
        pltpu.sync_copy(deg_hbm.at[:, pl.ds(base, CB)], idg)

        def deg_fill(s, p):
            buf = rows.at[p]

            def grp(g, carry):
                ixv = idg[s, pl.ds(g * 16, 16)]
                for l in range(16):
                    ix = ixv[l]
                    r = g * 16 + l
                    buf[r, pl.ds(0, 16)] = deg_tab[ix, pl.ds(0, 16)]
                    buf[r, pl.ds(16, 16)] = deg_tab[ix, pl.ds(16, 16)]
                return carry

            lax.fori_loop(0, CB // 16, grp, 0)

        def transpose(p):
            src = rows.at[p]
            dst = tbuf.at[p]
            lanes = lax.iota(jnp.int32, 16)
            cvecs = [(c0 + lanes) & (V_DIM - 1) for c0 in range(V_DIM)]

            def grp(g, carry):
                bvec = g * 16 + lanes
                for c0 in range(V_DIM):
                    v = plsc.load_gather(src, [bvec, cvecs[c0]])
                    plsc.store_scatter(dst, [cvecs[c0], bvec], v)
                return carry

            lax.fori_loop(0, CB // 16, grp, 0)

        def fire_adds(s, p):
            dst = rows.at[p]
            pltpu.async_copy(Wv.at[iv.at[s]], dst, sa[p], add=True)
            pltpu.async_copy(Wp.at[ip.at[s]], dst, sa[p], add=True)

        def wait_adds(s, p):
            dst = rows.at[p]
            pltpu.make_async_copy(Wv.at[iv.at[s]], dst, sa[p]).wait()
            pltpu.make_async_copy(Wp.at[ip.at[s]], dst, sa[p]).wait()

        def fire_wb(s, p):
            pltpu.async_copy(tbuf.at[p], out_hbm.at[s, :, pl.ds(base, CB)],
                             sw[p])

        def wait_wb(p):
            pltpu.make_async_copy(
                tbuf.at[p], out_hbm.at[0, :, pl.ds(base, CB)], sw[p]).wait()

        def step(s, u):
            p = u % NBUF
            q = (u + NBUF - 1) % NBUF
            @pl.when(s >= NBUF)
            def _():
                wait_wb(p)

            deg_fill(s, p)
            fire_adds(s, p)
            @pl.when(s >= 1)
            def _():
                wait_adds(s - 1, q)
                transpose(q)
                fire_wb(s - 1, q)

        def round_(g, carry):
            for u in range(NBUF):
                step(g * NBUF + u, u)
            return carry

        lax.fori_loop(0, L // NBUF, round_, 0)
        p_last = (L - 1) % NBUF
        wait_adds(L - 1, p_last)
        transpose(p_last)
        fire_wb(L - 1, p_last)
        for p in range(NBUF):
            wait_wb(p)

    return k


def kernel(vidx, pos, deg, W_vidx, W_pos, W_deg):
    B, L = vidx.shape
    out_t = _make_sc_kernel(B, L, W_deg.shape[0])(
        vidx.T, pos.T, deg.T, W_vidx, W_pos, W_deg)
    return jnp.transpose(out_t, (2, 0, 1))


# speedup vs baseline: 1.6367x; 1.0975x over previous
import functools
import jax
import jax.numpy as jnp
from jax import lax
from jax.experimental import pallas as pl
from jax.experimental.pallas import tpu as pltpu, tpu_sc as plsc

V_DIM = 32
NC, NS = 2, 16
NW = NC * NS
NBUF = 2
CB = 128


@functools.lru_cache(maxsize=None)
def _make_sc_kernel(B, L, DEG_ROWS):
    assert B == NW * CB
    mesh = plsc.VectorSubcoreMesh(core_axis_name="c", subcore_axis_name="s")

    @functools.partial(
        pl.kernel,
        out_type=jax.ShapeDtypeStruct((L, V_DIM, B), jnp.float32),
        mesh=mesh,
        scratch_types=[
            pltpu.VMEM((L, CB), jnp.int32),
            pltpu.VMEM((L, CB), jnp.int32),
            pltpu.VMEM((L, CB), jnp.int32),
            pltpu.VMEM((NBUF, CB, V_DIM), jnp.float32),
            pltpu.VMEM((NBUF, V_DIM, CB), jnp.float32),
            pltpu.VMEM((DEG_ROWS, V_DIM), jnp.float32),
            [pltpu.SemaphoreType.DMA] * NBUF,
            [pltpu.SemaphoreType.DMA] * NBUF,
        ],
        compiler_params=pltpu.CompilerParams(
            use_tc_tiling_on_sc=False, needs_layout_passes=False),
    )
    def k(vidx_hbm, pos_hbm, deg_hbm, Wv, Wp, Wd, out_hbm,
          iv, ip, idg, rows, tbuf, deg_tab, sa, sw):
        wid = lax.axis_index("s") * NC + lax.axis_index("c")
        base = wid * CB
        pltpu.sync_copy(Wd, deg_tab)
        pltpu.sync_copy(vidx_hbm.at[:, pl.ds(base, CB)], iv)
        pltpu.sync_copy(pos_hbm.at[:, pl.ds(base, CB)], ip)
        pltpu.sync_copy(deg_hbm.at[:, pl.ds(base, CB)], idg)

        def zero_rows(p):
            buf = rows.at[p]
            vz = jnp.zeros((16,), jnp.float32)

            def grp(g, carry):
                for j in range(8):
                    r = g * 4 + j // 2
                    buf[r, pl.ds((j % 2) * 16, 16)] = vz
                return carry

            lax.fori_loop(0, CB // 4, grp, 0)

        def transpose(s, p):
            src = rows.at[p]
            dst = tbuf.at[p]
            lanes = lax.iota(jnp.int32, 16)
            cvecs = [(c0 + lanes) & (V_DIM - 1) for c0 in range(V_DIM)]

            def grp(g, carry):
                bvec = g * 16 + lanes
                ixv = idg[s, pl.ds(g * 16, 16)]
                for c0 in range(V_DIM):
                    v = plsc.load_gather(src, [bvec, cvecs[c0]])
                    vd = plsc.load_gather(deg_tab, [ixv, cvecs[c0]])
                    plsc.store_scatter(dst, [cvecs[c0], bvec], v + vd)
                return carry

            lax.fori_loop(0, CB // 16, grp, 0)

        def fire_adds(s, p):
            dst = rows.at[p]
            pltpu.async_copy(Wv.at[iv.at[s]], dst, sa[p], add=True)
            pltpu.async_copy(Wp.at[ip.at[s]], dst, sa[p], add=True)

        def wait_adds(s, p):
            dst = rows.at[p]
            pltpu.make_async_copy(Wv.at[iv.at[s]], dst, sa[p]).wait()
            pltpu.make_async_copy(Wp.at[ip.at[s]], dst, sa[p]).wait()

        def fire_wb(s, p):
            pltpu.async_copy(tbuf.at[p], out_hbm.at[s, :, pl.ds(base, CB)],
                             sw[p])

        def wait_wb(p):
            pltpu.make_async_copy(
                tbuf.at[p], out_hbm.at[0, :, pl.ds(base, CB)], sw[p]).wait()

        def step(s, u):
            p = u % NBUF
            q = (u + NBUF - 1) % NBUF
            @pl.when(s >= NBUF)
            def _():
                wait_wb(p)

            zero_rows(p)
            fire_adds(s, p)
            @pl.when(s >= 1)
            def _():
                wait_adds(s - 1, q)
                transpose(s - 1, q)
                fire_wb(s - 1, q)

        def round_(g, carry):
            for u in range(NBUF):
                step(g * NBUF + u, u)
            return carry

        lax.fori_loop(0, L // NBUF, round_, 0)
        p_last = (L - 1) % NBUF
        wait_adds(L - 1, p_last)
        transpose(L - 1, p_last)
        fire_wb(L - 1, p_last)
        for p in range(NBUF):
            wait_wb(p)

    return k


def kernel(vidx, pos, deg, W_vidx, W_pos, W_deg):
    B, L = vidx.shape
    out_t = _make_sc_kernel(B, L, W_deg.shape[0])(
        vidx.T, pos.T, deg.T, W_vidx, W_pos, W_deg)
    return jnp.transpose(out_t, (2, 0, 1))
